# baseline (speedup 1.0000x reference)
import jax
import jax.numpy as jnp
from jax.experimental import pallas as pl
from jax.experimental.pallas import tpu as pltpu

Q = 1024
D = 64
B = 16
PPI = 64
KBLK = 10000


def _knn_kernel(qt_ref, qa_ref, m_hbm, patch_ref, img_ref,
                acc_ref, buf_ref, sems):
    k = pl.program_id(0)
    nk = pl.num_programs(0)
    slot = jax.lax.rem(k, 2)
    nslot = 1 - slot

    @pl.when(k == 0)
    def _():
        pltpu.make_async_copy(m_hbm.at[pl.ds(0, KBLK), :],
                              buf_ref.at[0], sems.at[0]).start()

    @pl.when(k + 1 < nk)
    def _():
        pltpu.make_async_copy(m_hbm.at[pl.ds((k + 1) * KBLK, KBLK), :],
                              buf_ref.at[nslot], sems.at[nslot]).start()

    pltpu.make_async_copy(m_hbm.at[pl.ds(k * KBLK, KBLK), :],
                          buf_ref.at[slot], sems.at[slot]).wait()
    qa = qa_ref[...]
    m = buf_ref[slot]

    mb = m.astype(jnp.bfloat16)
    msq = jax.lax.dot_general(mb * mb, jnp.ones((D, 1), jnp.bfloat16),
                              (((1,), (0,)), ((), ())),
                              preferred_element_type=jnp.float32)
    msq_hi = msq.astype(jnp.bfloat16)
    msq_lo = (msq - msq_hi.astype(jnp.float32)).astype(jnp.bfloat16)
    ma = jnp.concatenate([mb, msq_hi, msq_lo], axis=1)
    s = jax.lax.dot_general(ma, qa, (((1,), (0,)), ((), ())),
                            preferred_element_type=jnp.float32)
    blk_min = jnp.min(s, axis=0, keepdims=True)

    @pl.when(k == 0)
    def _():
        acc_ref[...] = blk_min

    @pl.when(k > 0)
    def _():
        acc_ref[...] = jnp.minimum(acc_ref[...], blk_min)

    @pl.when(k == nk - 1)
    def _():
        qt = qt_ref[...]
        qsq = jnp.sum(qt * qt, axis=0, keepdims=True)
        dmin = jnp.maximum(acc_ref[...] + qsq, 0.0)
        ps = jnp.sqrt(jnp.maximum(dmin, 1e-12))
        patch_ref[...] = ps
        qimg = jax.lax.broadcasted_iota(jnp.int32, (B, Q), 1) // PPI
        img = jax.lax.broadcasted_iota(jnp.int32, (B, Q), 0)
        masked = jnp.where(qimg == img, ps, -jnp.inf)
        img_ref[...] = jnp.max(masked, axis=1, keepdims=True)


def kernel(queries, memory_bank):
    K = memory_bank.shape[0]
    nblk = K // KBLK
    qt = queries.T
    qa = jnp.concatenate([qt * -2.0, jnp.ones((2, Q), jnp.float32)],
                         axis=0).astype(jnp.bfloat16)
    patch, img = pl.pallas_call(
        _knn_kernel,
        grid=(nblk,),
        in_specs=[
            pl.BlockSpec((D, Q), lambda k: (0, 0)),
            pl.BlockSpec((D + 2, Q), lambda k: (0, 0)),
            pl.BlockSpec(memory_space=pl.ANY),
        ],
        out_specs=[
            pl.BlockSpec((1, Q), lambda k: (0, 0)),
            pl.BlockSpec((B, 1), lambda k: (0, 0)),
        ],
        out_shape=[
            jax.ShapeDtypeStruct((1, Q), jnp.float32),
            jax.ShapeDtypeStruct((B, 1), jnp.float32),
        ],
        scratch_shapes=[
            pltpu.VMEM((1, Q), jnp.float32),
            pltpu.VMEM((2, KBLK, D), jnp.float32),
            pltpu.SemaphoreType.DMA((2,)),
        ],
    )(qt, qa, memory_bank)
    return patch.reshape(Q), img.reshape(B)


# speedup vs baseline: 1.0637x; 1.0637x over previous
import jax
import jax.numpy as jnp
from jax.experimental import pallas as pl
from jax.experimental.pallas import tpu as pltpu

Q = 1024
D = 64
B = 16
PPI = 64
KBLK = 10000


def _knn_kernel(qt_ref, qa_ref, m_hbm, patch_ref, img_ref,
                acc_ref, buf_ref, sems):
    k = pl.program_id(0)
    nk = pl.num_programs(0)
    slot = jax.lax.rem(k, 2)
    nslot = 1 - slot

    @pl.when(k == 0)
    def _():
        pltpu.make_async_copy(m_hbm.at[pl.ds(0, KBLK), :],
                              buf_ref.at[0], sems.at[0]).start()

    @pl.when(k + 1 < nk)
    def _():
        pltpu.make_async_copy(m_hbm.at[pl.ds((k + 1) * KBLK, KBLK), :],
                              buf_ref.at[nslot], sems.at[nslot]).start()

    pltpu.make_async_copy(m_hbm.at[pl.ds(k * KBLK, KBLK), :],
                          buf_ref.at[slot], sems.at[slot]).wait()
    qa = qa_ref[...]
    m = buf_ref[slot]

    msq = jax.lax.dot_general(m * m, jnp.ones((D, 1), jnp.float32),
                              (((1,), (0,)), ((), ())),
                              preferred_element_type=jnp.float32)
    ma = jnp.concatenate([m, msq], axis=1)
    s = jax.lax.dot_general(ma, qa, (((1,), (0,)), ((), ())),
                            preferred_element_type=jnp.float32)
    blk_min = jnp.min(s, axis=0, keepdims=True)

    @pl.when(k == 0)
    def _():
        acc_ref[...] = blk_min

    @pl.when(k > 0)
    def _():
        acc_ref[...] = jnp.minimum(acc_ref[...], blk_min)

    @pl.when(k == nk - 1)
    def _():
        qt = qt_ref[...]
        qsq = jnp.sum(qt * qt, axis=0, keepdims=True)
        dmin = jnp.maximum(acc_ref[...] + qsq, 0.0)
        ps = jnp.sqrt(jnp.maximum(dmin, 1e-12))
        patch_ref[...] = ps
        qimg = jax.lax.broadcasted_iota(jnp.int32, (B, Q), 1) // PPI
        img = jax.lax.broadcasted_iota(jnp.int32, (B, Q), 0)
        masked = jnp.where(qimg == img, ps, -jnp.inf)
        img_ref[...] = jnp.max(masked, axis=1, keepdims=True)


def kernel(queries, memory_bank):
    K = memory_bank.shape[0]
    nblk = K // KBLK
    qt = queries.T
    qa = jnp.concatenate([qt * -2.0, jnp.ones((1, Q), jnp.float32)],
                         axis=0)
    patch, img = pl.pallas_call(
        _knn_kernel,
        grid=(nblk,),
        in_specs=[
            pl.BlockSpec((D, Q), lambda k: (0, 0)),
            pl.BlockSpec((D + 1, Q), lambda k: (0, 0)),
            pl.BlockSpec(memory_space=pl.ANY),
        ],
        out_specs=[
            pl.BlockSpec((1, Q), lambda k: (0, 0)),
            pl.BlockSpec((B, 1), lambda k: (0, 0)),
        ],
        out_shape=[
            jax.ShapeDtypeStruct((1, Q), jnp.float32),
            jax.ShapeDtypeStruct((B, 1), jnp.float32),
        ],
        scratch_shapes=[
            pltpu.VMEM((1, Q), jnp.float32),
            pltpu.VMEM((2, KBLK, D), jnp.float32),
            pltpu.SemaphoreType.DMA((2,)),
        ],
    )(qt, qa, memory_bank)
    return patch.reshape(Q), img.reshape(B)
